# Initial kernel scaffold; baseline (speedup 1.0000x reference)
import functools

import jax
import jax.numpy as jnp
from jax import lax
from jax.experimental import pallas as pl
from jax.experimental.pallas import tpu as pltpu
from jax.experimental.pallas import tpu_sc as plsc

N = 10000
H = 128
E = 320000
EL = 100000
CW = 16

NC, NS = 2, 16
NW = NC * NS
ROWS = E // 128
RPW = ROWS // NW
NPT = N // NS

_MESH = plsc.VectorSubcoreMesh(core_axis_name="c", subcore_axis_name="s")


def _make_segsum(with_counts):
  out_type = [jax.ShapeDtypeStruct((NC, N, H), jnp.float32)]
  scratch = [
      pltpu.VMEM((RPW + 1, 128), jnp.int32),
      pltpu.VMEM((RPW + 1, 128), jnp.int32),
      pltpu.VMEM((128, H), jnp.float32),
      pltpu.VMEM((128, H), jnp.float32),
      pltpu.VMEM((25, H), jnp.float32),
      pltpu.VMEM_SHARED((N, H), jnp.float32),
      pltpu.SemaphoreType.DMA,
      pltpu.SemaphoreType.DMA,
  ]
  if with_counts:
    out_type.append(jax.ShapeDtypeStruct((NC, N, CW), jnp.float32))
    scratch += [
        pltpu.VMEM((128, CW), jnp.float32),
        pltpu.VMEM((25, CW), jnp.float32),
        pltpu.VMEM_SHARED((N, CW), jnp.float32),
    ]

  @functools.partial(pl.kernel, out_type=tuple(out_type), mesh=_MESH,
                     scratch_types=scratch)
  def seg(*refs):
    if with_counts:
      (src2d, dst2d, table, acc_out, cnt_out,
       idx_all, dst_all, rows0, rows1, zbuf, acc_s, sem0, sem1,
       ones_v, zcnt, cnt_s) = refs
    else:
      (src2d, dst2d, table, acc_out,
       idx_all, dst_all, rows0, rows1, zbuf, acc_s, sem0, sem1) = refs

    c_id = lax.axis_index("c")
    s_id = lax.axis_index("s")
    w = c_id * NS + s_id
    base = s_id * NPT

    z16 = jnp.zeros((16,), jnp.float32)
    for r in range(25):
      for k in range(H // 16):
        zbuf[r, pl.ds(k * 16, 16)] = z16
    if with_counts:
      for r in range(25):
        zcnt[r, pl.ds(0, CW)] = z16
      o16 = jnp.ones((16,), jnp.float32)
      for r in range(128):
        ones_v[r, pl.ds(0, CW)] = o16

    def zloop(j, carry):
      pltpu.sync_copy(zbuf, acc_s.at[pl.ds(base + j * 25, 25)])
      if with_counts:
        pltpu.sync_copy(zcnt, cnt_s.at[pl.ds(base + j * 25, 25)])
      return carry
    lax.fori_loop(0, NPT // 25, zloop, 0)

    lo = w * RPW
    pltpu.sync_copy(src2d.at[pl.ds(lo, RPW)], idx_all.at[pl.ds(0, RPW)])
    pltpu.sync_copy(dst2d.at[pl.ds(lo, RPW)], dst_all.at[pl.ds(0, RPW)])

    @pl.when(w < ROWS - NW * RPW)
    def _():
      pltpu.sync_copy(src2d.at[pl.ds(NW * RPW + w, 1)], idx_all.at[pl.ds(RPW, 1)])
      pltpu.sync_copy(dst2d.at[pl.ds(NW * RPW + w, 1)], dst_all.at[pl.ds(RPW, 1)])

    plsc.subcore_barrier()

    def scat(rows, j):
      pltpu.sync_copy(rows, acc_s.at[dst_all.at[j]], add=True)
      if with_counts:
        pltpu.sync_copy(ones_v, cnt_s.at[dst_all.at[j]], add=True)

    pltpu.async_copy(table.at[idx_all.at[0]], rows0, sem0)

    def body(i, carry):
      j0 = 2 * i
      pltpu.async_copy(table.at[idx_all.at[j0 + 1]], rows1, sem1)
      pltpu.make_async_copy(table.at[idx_all.at[j0]], rows0, sem0).wait()
      scat(rows0, j0)

      @pl.when(i < RPW // 2 - 1)
      def _():
        pltpu.async_copy(table.at[idx_all.at[j0 + 2]], rows0, sem0)
      pltpu.make_async_copy(table.at[idx_all.at[j0 + 1]], rows1, sem1).wait()
      scat(rows1, j0 + 1)
      return carry
    lax.fori_loop(0, RPW // 2, body, 0)

    @pl.when(w < ROWS - NW * RPW)
    def _():
      pltpu.async_copy(table.at[idx_all.at[RPW]], rows0, sem0).wait()
      scat(rows0, RPW)

    plsc.subcore_barrier()
    pltpu.sync_copy(acc_s.at[pl.ds(base, NPT)],
                    acc_out.at[c_id, pl.ds(base, NPT)])
    if with_counts:
      pltpu.sync_copy(cnt_s.at[pl.ds(base, NPT)],
                      cnt_out.at[c_id, pl.ds(base, NPT)])

  return seg


_segsum_cnt = _make_segsum(True)
_segsum = _make_segsum(False)

_DEC_FULL = EL // 128
_DEC_TAIL = EL - _DEC_FULL * 128


@functools.partial(
    pl.kernel,
    out_type=jax.ShapeDtypeStruct((EL,), jnp.float32),
    mesh=_MESH,
    scratch_types=[
        pltpu.VMEM((128,), jnp.int32),
        pltpu.VMEM((128,), jnp.int32),
        pltpu.VMEM((_DEC_TAIL,), jnp.int32),
        pltpu.VMEM((_DEC_TAIL,), jnp.int32),
        pltpu.VMEM((128, H), jnp.float32),
        pltpu.VMEM((128, H), jnp.float32),
        pltpu.VMEM((128,), jnp.float32),
        pltpu.SemaphoreType.DMA,
        pltpu.SemaphoreType.DMA,
    ])
def _decoder(el0, el1, oa, ob, res, ia, ib, ia_t, ib_t, ra, rb, rv, sa, sb):
  c_id = lax.axis_index("c")
  s_id = lax.axis_index("s")
  w = c_id * NS + s_id

  def rowdot(n):
    def row(r, carry):
      acc = ra[r, pl.ds(0, 16)] * rb[r, pl.ds(0, 16)]
      for k in range(1, H // 16):
        acc = acc + ra[r, pl.ds(16 * k, 16)] * rb[r, pl.ds(16 * k, 16)]
      rv[r] = jnp.sum(acc)
      return carry
    lax.fori_loop(0, n, row, 0)

  def chunk(i, carry):
    c = i * NW + w

    @pl.when(c < _DEC_FULL)
    def _():
      b0 = c * 128
      pltpu.sync_copy(el0.at[pl.ds(b0, 128)], ia)
      pltpu.sync_copy(el1.at[pl.ds(b0, 128)], ib)
      da = pltpu.async_copy(oa.at[ia], ra, sa)
      db = pltpu.async_copy(ob.at[ib], rb, sb)
      da.wait()
      db.wait()
      rowdot(128)
      pltpu.sync_copy(rv, res.at[pl.ds(b0, 128)])

    @pl.when(c == _DEC_FULL)
    def _():
      b0 = _DEC_FULL * 128
      pltpu.sync_copy(el0.at[pl.ds(b0, _DEC_TAIL)], ia_t)
      pltpu.sync_copy(el1.at[pl.ds(b0, _DEC_TAIL)], ib_t)
      da = pltpu.async_copy(oa.at[ia_t], ra.at[pl.ds(0, _DEC_TAIL)], sa)
      db = pltpu.async_copy(ob.at[ib_t], rb.at[pl.ds(0, _DEC_TAIL)], sb)
      da.wait()
      db.wait()
      rowdot(_DEC_TAIL)
      pltpu.sync_copy(rv.at[pl.ds(0, _DEC_TAIL)], res.at[pl.ds(b0, _DEC_TAIL)])
    return carry

  lax.fori_loop(0, _DEC_FULL // NW + 1, chunk, 0)


def _tc_layer(relu):
  BN = 1000

  def body(acc_ref, cnt_ref, x_ref, wl_ref, wr_ref, b_ref, o_ref):
    s = acc_ref[0] + acc_ref[1]
    c = cnt_ref[0, :, 0:1] + cnt_ref[1, :, 0:1]
    mean = s / jnp.maximum(c, 1.0)
    o = (jnp.dot(mean, wl_ref[...], preferred_element_type=jnp.float32)
         + b_ref[...]
         + jnp.dot(x_ref[...], wr_ref[...], preferred_element_type=jnp.float32))
    o_ref[...] = jnp.maximum(o, 0.0) if relu else o

  return pl.pallas_call(
      body,
      grid=(N // BN,),
      in_specs=[
          pl.BlockSpec((NC, BN, H), lambda i: (0, i, 0)),
          pl.BlockSpec((NC, BN, CW), lambda i: (0, i, 0)),
          pl.BlockSpec((BN, H), lambda i: (i, 0)),
          pl.BlockSpec((H, H), lambda i: (0, 0)),
          pl.BlockSpec((H, H), lambda i: (0, 0)),
          pl.BlockSpec((1, H), lambda i: (0, 0)),
      ],
      out_specs=pl.BlockSpec((BN, H), lambda i: (i, 0)),
      out_shape=jax.ShapeDtypeStruct((N, H), jnp.float32),
  )


_tc_relu = _tc_layer(True)
_tc_lin = _tc_layer(False)


def kernel(src_node_id, tgt_node_id, edge_index_fwd, edge_index_rev,
           edge_label_index, emb_src, emb_tgt, W1f_l, W1f_r, W1r_l, W1r_r,
           W2f_l, W2f_r, W2r_l, W2r_r, b1f, b1r, b2f, b2r):
  src_f = edge_index_fwd[0].reshape(ROWS, 128)
  dst_f = edge_index_fwd[1].reshape(ROWS, 128)
  src_r = edge_index_rev[0].reshape(ROWS, 128)
  dst_r = edge_index_rev[1].reshape(ROWS, 128)
  el0 = edge_label_index[0]
  el1 = edge_label_index[1]

  acc1f, cnt_f = _segsum_cnt(src_f, dst_f, emb_src)
  acc1r, cnt_r = _segsum_cnt(src_r, dst_r, emb_tgt)

  h_tgt = _tc_relu(acc1f, cnt_f, emb_tgt, W1f_l, W1f_r, b1f.reshape(1, H))
  h_src = _tc_relu(acc1r, cnt_r, emb_src, W1r_l, W1r_r, b1r.reshape(1, H))

  (acc2f,) = _segsum(src_f, dst_f, h_src)
  (acc2r,) = _segsum(src_r, dst_r, h_tgt)

  o_tgt = _tc_lin(acc2f, cnt_f, h_tgt, W2f_l, W2f_r, b2f.reshape(1, H))
  o_src = _tc_lin(acc2r, cnt_r, h_src, W2r_l, W2r_r, b2r.reshape(1, H))

  return _decoder(el0, el1, o_src, o_tgt)


# speedup vs baseline: 3.6443x; 3.6443x over previous
import functools

import jax
import jax.numpy as jnp
from jax import lax
from jax.experimental import pallas as pl
from jax.experimental.pallas import tpu as pltpu
from jax.experimental.pallas import tpu_sc as plsc

N = 10000
H = 128
E = 320000
EL = 100000
CW = 16

NC, NS = 2, 16
NW = NC * NS
ROWS = 2560
RPT = ROWS // NS
G = 16
NG = RPT // G
NPT = 624
TRASH = N
EL_PAD = 102400
DPW = EL_PAD // 128 // NW

_MESH = plsc.VectorSubcoreMesh(core_axis_name="c", subcore_axis_name="s")


def _writeout(s_id, shared, out, width):
  @pl.when(s_id < NS - 1)
  def _():
    base = s_id * NPT
    pltpu.sync_copy(shared.at[pl.ds(base, NPT)], out.at[pl.ds(base, NPT)])

  @pl.when(s_id == NS - 1)
  def _():
    last = (NS - 1) * NPT
    pltpu.sync_copy(shared.at[pl.ds(last, N - last)],
                    out.at[pl.ds(last, N - last)])


@functools.partial(
    pl.kernel,
    out_type=(jax.ShapeDtypeStruct((N, H), jnp.float32),
              jax.ShapeDtypeStruct((N, H), jnp.float32)),
    mesh=_MESH,
    scratch_types=[
        pltpu.VMEM((G, 128), jnp.int32),
        pltpu.VMEM((G, 128), jnp.int32),
        pltpu.VMEM((128, H), jnp.float32),
        pltpu.VMEM((128, H), jnp.float32),
        pltpu.VMEM_SHARED((N + 8, H), jnp.float32),
        pltpu.SemaphoreType.DMA,
        pltpu.SemaphoreType.DMA,
    ])
def _segsum(src_f, dst_f, table_f, src_r, dst_r, table_r, acc_f, acc_r,
            idx_all, dst_all, rows0, rows1, acc_s, sem0, sem1):
  c_id = lax.axis_index("c")
  s_id = lax.axis_index("s")
  base = s_id * NPT

  z16 = jnp.zeros((16,), jnp.float32)

  def zfill(r, carry):
    for k in range(H // 16):
      rows0[r, pl.ds(k * 16, 16)] = z16
    return carry
  lax.fori_loop(0, 128, zfill, 0)

  def zloop(j, carry):
    pltpu.sync_copy(rows0, acc_s.at[pl.ds(base + j * 128, 128)])
    return carry
  lax.fori_loop(0, 5, zloop, 0)

  plsc.subcore_barrier()

  def run(src2d, dst2d, table):
    lo = s_id * RPT

    def group(g, carry):
      g0 = lo + g * G
      pltpu.sync_copy(src2d.at[pl.ds(g0, G)], idx_all)
      pltpu.sync_copy(dst2d.at[pl.ds(g0, G)], dst_all)
      pltpu.async_copy(table.at[idx_all.at[0]], rows0, sem0)

      def body(i, carry2):
        j0 = 2 * i
        pltpu.async_copy(table.at[idx_all.at[j0 + 1]], rows1, sem1)
        pltpu.make_async_copy(table.at[idx_all.at[j0]], rows0, sem0).wait()
        pltpu.sync_copy(rows0, acc_s.at[dst_all.at[j0]], add=True)

        @pl.when(i < G // 2 - 1)
        def _():
          pltpu.async_copy(table.at[idx_all.at[j0 + 2]], rows0, sem0)
        pltpu.make_async_copy(table.at[idx_all.at[j0 + 1]], rows1, sem1).wait()
        pltpu.sync_copy(rows1, acc_s.at[dst_all.at[j0 + 1]], add=True)
        return carry2
      lax.fori_loop(0, G // 2, body, 0)
      return carry
    lax.fori_loop(0, NG, group, 0)

  @pl.when(c_id == 0)
  def _():
    run(src_f, dst_f, table_f)

  @pl.when(c_id == 1)
  def _():
    run(src_r, dst_r, table_r)

  plsc.subcore_barrier()

  @pl.when(c_id == 0)
  def _():
    _writeout(s_id, acc_s, acc_f, H)

  @pl.when(c_id == 1)
  def _():
    _writeout(s_id, acc_s, acc_r, H)


@functools.partial(
    pl.kernel,
    out_type=(jax.ShapeDtypeStruct((N, H), jnp.float32),
              jax.ShapeDtypeStruct((N, H), jnp.float32)),
    mesh=_MESH,
    scratch_types=[
        pltpu.VMEM((G, 128), jnp.int32),
        pltpu.VMEM((128, H), jnp.float32),
        pltpu.VMEM_SHARED((N + 8, H), jnp.float32),
    ])
def _counts(dst_f, dst_r, cnt_f, cnt_r, dst_all, ones_v, cnt_s):
  c_id = lax.axis_index("c")
  s_id = lax.axis_index("s")
  base = s_id * NPT

  z16 = jnp.zeros((16,), jnp.float32)
  o16 = jnp.ones((16,), jnp.float32)

  def zfill(r, carry):
    for k in range(H // 16):
      ones_v[r, pl.ds(k * 16, 16)] = z16
    return carry
  lax.fori_loop(0, 128, zfill, 0)

  def zloop(j, carry):
    pltpu.sync_copy(ones_v, cnt_s.at[pl.ds(base + j * 128, 128)])
    return carry
  lax.fori_loop(0, 5, zloop, 0)

  def ofill(r, carry):
    for k in range(H // 16):
      ones_v[r, pl.ds(k * 16, 16)] = o16
    return carry
  lax.fori_loop(0, 128, ofill, 0)

  plsc.subcore_barrier()

  def run(dst2d):
    lo = s_id * RPT

    def group(g, carry):
      pltpu.sync_copy(dst2d.at[pl.ds(lo + g * G, G)], dst_all)

      def body(j, carry2):
        pltpu.sync_copy(ones_v, cnt_s.at[dst_all.at[j]], add=True)
        return carry2
      lax.fori_loop(0, G, body, 0)
      return carry
    lax.fori_loop(0, NG, group, 0)

  @pl.when(c_id == 0)
  def _():
    run(dst_f)

  @pl.when(c_id == 1)
  def _():
    run(dst_r)

  plsc.subcore_barrier()

  @pl.when(c_id == 0)
  def _():
    _writeout(s_id, cnt_s, cnt_f, H)

  @pl.when(c_id == 1)
  def _():
    _writeout(s_id, cnt_s, cnt_r, H)


@functools.partial(
    pl.kernel,
    out_type=jax.ShapeDtypeStruct((EL_PAD, 16), jnp.float32),
    mesh=_MESH,
    scratch_types=[
        pltpu.VMEM((128,), jnp.int32),
        pltpu.VMEM((128,), jnp.int32),
        pltpu.VMEM((128, H), jnp.float32),
        pltpu.VMEM((128, H), jnp.float32),
        pltpu.VMEM((128, 16), jnp.float32),
        pltpu.SemaphoreType.DMA,
        pltpu.SemaphoreType.DMA,
    ])
def _decoder(el0, el1, oa, ob, res, ia, ib, ra, rb, rv, sa, sb):
  c_id = lax.axis_index("c")
  s_id = lax.axis_index("s")
  w = c_id * NS + s_id

  def rowdot():
    def row(r, carry):
      acc = ra[r, pl.ds(0, 16)] * rb[r, pl.ds(0, 16)]
      for k in range(1, H // 16):
        acc = acc + ra[r, pl.ds(16 * k, 16)] * rb[r, pl.ds(16 * k, 16)]
      rv[r, pl.ds(0, 16)] = acc
      return carry
    lax.fori_loop(0, 128, row, 0)

  def chunk(i, carry):
    b0 = (w * DPW + i) * 128
    pltpu.sync_copy(el0.at[pl.ds(b0, 128)], ia)
    pltpu.sync_copy(el1.at[pl.ds(b0, 128)], ib)
    da = pltpu.async_copy(oa.at[ia], ra, sa)
    db = pltpu.async_copy(ob.at[ib], rb, sb)
    da.wait()
    db.wait()
    rowdot()
    pltpu.sync_copy(rv, res.at[pl.ds(b0, 128)])
    return carry

  lax.fori_loop(0, DPW, chunk, 0)


def _lane_sum():
  BR = EL_PAD // 128 // 4

  def body(p_ref, o_ref):
    o_ref[...] = jnp.sum(p_ref[...], axis=2)

  return pl.pallas_call(
      body,
      grid=(4,),
      in_specs=[pl.BlockSpec((BR, 128, 16), lambda i: (i, 0, 0))],
      out_specs=pl.BlockSpec((BR, 128), lambda i: (i, 0)),
      out_shape=jax.ShapeDtypeStruct((EL_PAD // 128, 128), jnp.float32),
  )


_lane_sum_call = _lane_sum()


def _tc_layer(relu):
  BN = 1000

  def body(af_ref, cf_ref, xf_ref, wfl_ref, wfr_ref, bf_ref,
           ar_ref, cr_ref, xr_ref, wrl_ref, wrr_ref, br_ref,
           of_ref, or_ref):
    def one(a, c, x, wl, wr, b):
      mean = a / jnp.maximum(c[:, 0:1], 1.0)
      o = (jnp.dot(mean, wl, preferred_element_type=jnp.float32) + b
           + jnp.dot(x, wr, preferred_element_type=jnp.float32))
      return jnp.maximum(o, 0.0) if relu else o

    of_ref[...] = one(af_ref[...], cf_ref[...], xf_ref[...],
                      wfl_ref[...], wfr_ref[...], bf_ref[...])
    or_ref[...] = one(ar_ref[...], cr_ref[...], xr_ref[...],
                      wrl_ref[...], wrr_ref[...], br_ref[...])

  row_spec = pl.BlockSpec((BN, H), lambda i: (i, 0))
  cnt_spec = pl.BlockSpec((BN, H), lambda i: (i, 0))
  w_spec = pl.BlockSpec((H, H), lambda i: (0, 0))
  b_spec = pl.BlockSpec((1, H), lambda i: (0, 0))
  return pl.pallas_call(
      body,
      grid=(N // BN,),
      in_specs=[row_spec, cnt_spec, row_spec, w_spec, w_spec, b_spec,
                row_spec, cnt_spec, row_spec, w_spec, w_spec, b_spec],
      out_specs=[row_spec, row_spec],
      out_shape=[jax.ShapeDtypeStruct((N, H), jnp.float32),
                 jax.ShapeDtypeStruct((N, H), jnp.float32)],
  )


_tc_relu = _tc_layer(True)
_tc_lin = _tc_layer(False)


def kernel(src_node_id, tgt_node_id, edge_index_fwd, edge_index_rev,
           edge_label_index, emb_src, emb_tgt, W1f_l, W1f_r, W1r_l, W1r_r,
           W2f_l, W2f_r, W2r_l, W2r_r, b1f, b1r, b2f, b2r):
  ep = ROWS * 128 - E
  pad_src = jnp.zeros((ep,), jnp.int32)
  pad_dst = jnp.full((ep,), TRASH, jnp.int32)
  src_f = jnp.concatenate([edge_index_fwd[0], pad_src]).reshape(ROWS, 128)
  dst_f = jnp.concatenate([edge_index_fwd[1], pad_dst]).reshape(ROWS, 128)
  src_r = jnp.concatenate([edge_index_rev[0], pad_src]).reshape(ROWS, 128)
  dst_r = jnp.concatenate([edge_index_rev[1], pad_dst]).reshape(ROWS, 128)
  pad_el = jnp.zeros((EL_PAD - EL,), jnp.int32)
  el0 = jnp.concatenate([edge_label_index[0], pad_el])
  el1 = jnp.concatenate([edge_label_index[1], pad_el])

  cnt_f, cnt_r = _counts(dst_f, dst_r)
  acc1f, acc1r = _segsum(src_f, dst_f, emb_src, src_r, dst_r, emb_tgt)

  h_tgt, h_src = _tc_relu(acc1f, cnt_f, emb_tgt, W1f_l, W1f_r,
                          b1f.reshape(1, H),
                          acc1r, cnt_r, emb_src, W1r_l, W1r_r,
                          b1r.reshape(1, H))

  acc2f, acc2r = _segsum(src_f, dst_f, h_src, src_r, dst_r, h_tgt)

  o_tgt, o_src = _tc_lin(acc2f, cnt_f, h_tgt, W2f_l, W2f_r,
                         b2f.reshape(1, H),
                         acc2r, cnt_r, h_src, W2r_l, W2r_r,
                         b2r.reshape(1, H))

  partial = _decoder(el0, el1, o_src, o_tgt)
  out2d = _lane_sum_call(partial.reshape(EL_PAD // 128, 128, 16))
  return out2d.reshape(EL_PAD)[:EL]
